# Initial kernel scaffold; baseline (speedup 1.0000x reference)
import functools

import jax
import jax.numpy as jnp
from jax import lax
from jax.experimental import pallas as pl
from jax.experimental.pallas import tpu as pltpu
from jax.experimental.pallas import tpu_sc as plsc

N_NODES = 10000
D = 128

_NC = 2
_NS = 16
_NW = _NC * _NS
_C = 80
_CH = 125
_EDGE_ROWS = 4000
_RPT = N_NODES // _NS
_ZR = 125


def _mm_body(x_ref, w_ref, o_ref):
    o_ref[...] = jnp.dot(x_ref[...], w_ref[...],
                         preferred_element_type=jnp.float32)


def _mm2_body(a_ref, b_ref, w_ref, o_ref):
    o_ref[...] = jnp.dot(a_ref[...] + b_ref[...], w_ref[...],
                         preferred_element_type=jnp.float32)


def _add_body(a_ref, b_ref, o_ref):
    o_ref[...] = a_ref[...] + b_ref[...]


_BM = 1000


def _row_spec():
    return pl.BlockSpec((_BM, D), lambda i: (i, 0))


def _w_spec():
    return pl.BlockSpec((D, D), lambda i: (0, 0))


def _tc_mm(x, W):
    return pl.pallas_call(
        _mm_body,
        grid=(N_NODES // _BM,),
        in_specs=[_row_spec(), _w_spec()],
        out_specs=_row_spec(),
        out_shape=jax.ShapeDtypeStruct((N_NODES, D), jnp.float32),
    )(x, W)


def _tc_mm2(a, b, W):
    return pl.pallas_call(
        _mm2_body,
        grid=(N_NODES // _BM,),
        in_specs=[_row_spec(), _row_spec(), _w_spec()],
        out_specs=_row_spec(),
        out_shape=jax.ShapeDtypeStruct((N_NODES, D), jnp.float32),
    )(a, b, W)


def _tc_add(a, b):
    return pl.pallas_call(
        _add_body,
        grid=(N_NODES // _BM,),
        in_specs=[_row_spec(), _row_spec()],
        out_specs=_row_spec(),
        out_shape=jax.ShapeDtypeStruct((N_NODES, D), jnp.float32),
    )(a, b)


def _sc_agg_body(h_hbm, src_hbm, dst_hbm, w_hbm, out_hbm,
                 src_v, dst_v, w_v, rows_v, z_v, acc, sem):
    cid = lax.axis_index("c")
    sid = lax.axis_index("s")
    wid = cid * _NS + sid

    zv = jnp.zeros((16,), jnp.float32)

    def zero_row(r, carry):
        for g in range(D // 16):
            z_v[r, pl.ds(g * 16, 16)] = zv
        return carry

    lax.fori_loop(0, _ZR, zero_row, 0)
    for b in range(_RPT // _ZR):
        pltpu.sync_copy(z_v, acc.at[pl.ds(sid * _RPT + b * _ZR, _ZR)])
    plsc.subcore_barrier()

    base = wid * _CH
    pltpu.sync_copy(src_hbm.at[pl.ds(base, _CH)], src_v)
    pltpu.sync_copy(dst_hbm.at[pl.ds(base, _CH)], dst_v)
    pltpu.sync_copy(w_hbm.at[pl.ds(base, _CH)], w_v)

    def chunk(j, carry):
        pltpu.async_copy(h_hbm.at[src_v.at[j]], rows_v, sem).wait()

        def edge(e, c2):
            wv = jnp.full((16,), w_v[j, e], jnp.float32)
            for g in range(D // 16):
                s = (e, pl.ds(g * 16, 16))
                rows_v[s] = rows_v[s] * wv
            return c2

        lax.fori_loop(0, _C, edge, 0)

        pltpu.sync_copy(rows_v, acc.at[dst_v.at[j]], add=True)
        return carry

    lax.fori_loop(0, _CH, chunk, 0)

    plsc.subcore_barrier()
    for c in range(_NC):
        @pl.when(cid == c)
        def _dump(c=c):
            for b in range(_RPT // _ZR):
                r0 = sid * _RPT + b * _ZR
                pltpu.sync_copy(acc.at[pl.ds(r0, _ZR)],
                                out_hbm.at[c, pl.ds(r0, _ZR)])


def _sc_agg(h, src2d, dst2d, w2d):
    mesh = plsc.VectorSubcoreMesh(core_axis_name="c", subcore_axis_name="s")
    f = pl.kernel(
        _sc_agg_body,
        out_type=jax.ShapeDtypeStruct((_NC, N_NODES, D), jnp.float32),
        mesh=mesh,
        scratch_types=[
            pltpu.VMEM((_CH, _C), jnp.int32),
            pltpu.VMEM((_CH, _C), jnp.int32),
            pltpu.VMEM((_CH, _C), jnp.float32),
            pltpu.VMEM((_C, D), jnp.float32),
            pltpu.VMEM((_ZR, D), jnp.float32),
            pltpu.VMEM_SHARED((N_NODES, D), jnp.float32),
            pltpu.SemaphoreType.DMA,
        ],
    )
    return f(h, src2d, dst2d, w2d)


def kernel(x, edge_index, edge_weight, W1, W2):
    src = edge_index[0].astype(jnp.int32).reshape(_EDGE_ROWS, _C)
    dst = edge_index[1].astype(jnp.int32).reshape(_EDGE_ROWS, _C)
    w = edge_weight.reshape(_EDGE_ROWS, _C)

    h1 = _tc_mm(x, W1)
    p = _sc_agg(h1, src, dst, w)
    h2 = _tc_mm2(p[0], p[1], W2)
    q = _sc_agg(h2, src, dst, w)
    return _tc_add(q[0], q[1])


# speedup vs baseline: 3.0590x; 3.0590x over previous
import functools

import jax
import jax.numpy as jnp
from jax import lax
from jax.experimental import pallas as pl
from jax.experimental.pallas import tpu as pltpu
from jax.experimental.pallas import tpu_sc as plsc

N_NODES = 10000
D = 128

_NC = 2
_NS = 16
_C = 128
_CH = 157
_EPT = 20000
_EPT_PAD = _C * _CH

_NPC = N_NODES // _NC
_DUMMY = _NPC
_ACC_ROWS = 5024

_RPT = 312
_TAIL = 8
_ZTAIL = 32


def _fused_mm_body(u_ref, w1_ref, w2_ref, o_ref):
    wc = jnp.dot(w1_ref[...], w2_ref[...], preferred_element_type=jnp.float32)
    o_ref[...] = jnp.dot(u_ref[...], wc, preferred_element_type=jnp.float32)


_BM = 1000


def _tc_fused_mm(u, W1, W2):
    return pl.pallas_call(
        _fused_mm_body,
        grid=(N_NODES // _BM,),
        in_specs=[
            pl.BlockSpec((_BM, D), lambda i: (i, 0)),
            pl.BlockSpec((D, D), lambda i: (0, 0)),
            pl.BlockSpec((D, D), lambda i: (0, 0)),
        ],
        out_specs=pl.BlockSpec((_BM, D), lambda i: (i, 0)),
        out_shape=jax.ShapeDtypeStruct((N_NODES, D), jnp.float32),
    )(u, W1, W2)


def _sc_agg_body(h_hbm, src_hbm, dst_hbm, w_hbm, zeros_hbm, out_hbm,
                 src_v, dst_v, w_v, rows_v, dloc_v, acc, sem):
    cid = lax.axis_index("c")
    sid = lax.axis_index("s")

    pltpu.sync_copy(zeros_hbm.at[pl.ds(sid * _RPT, _RPT)],
                    acc.at[pl.ds(sid * _RPT, _RPT)])

    @pl.when(sid == _NS - 1)
    def _zero_tail():
        pltpu.sync_copy(zeros_hbm.at[pl.ds(_NS * _RPT, _ZTAIL)],
                        acc.at[pl.ds(_NS * _RPT, _ZTAIL)])

    pltpu.sync_copy(src_hbm.at[sid], src_v)
    pltpu.sync_copy(dst_hbm.at[sid], dst_v)
    pltpu.sync_copy(w_hbm.at[sid], w_v)
    plsc.subcore_barrier()

    base = cid * _NPC
    basev = jnp.full((16,), base, jnp.int32)
    limv = jnp.full((16,), _NPC, jnp.int32)
    dummyv = jnp.full((16,), _DUMMY, jnp.int32) + lax.iota(jnp.int32, 16)

    def chunk(j, carry):
        pltpu.async_copy(h_hbm.at[src_v.at[j]], rows_v, sem).wait()

        def group(k, c2):
            wvec = w_v[j, pl.ds(k * 16, 16)]
            dv = dst_v[j, pl.ds(k * 16, 16)] - basev
            ok = (dv >= 0) & (dv < limv)
            dloc_v[pl.ds(k * 16, 16)] = jnp.where(ok, dv, dummyv)
            for l in range(16):
                wv = jnp.full((16,), wvec[l], jnp.float32)
                e = k * 16 + l
                for g in range(D // 16):
                    s = (e, pl.ds(g * 16, 16))
                    rows_v[s] = rows_v[s] * wv
            return c2

        lax.fori_loop(0, _C // 16, group, 0)
        pltpu.sync_copy(rows_v, acc.at[dloc_v], add=True)
        return carry

    lax.fori_loop(0, _CH, chunk, 0)
    plsc.subcore_barrier()

    for c in range(_NC):
        @pl.when(cid == c)
        def _dump(c=c):
            pltpu.sync_copy(acc.at[pl.ds(sid * _RPT, _RPT)],
                            out_hbm.at[c, pl.ds(sid * _RPT, _RPT)])

            @pl.when(sid == _NS - 1)
            def _dump_tail():
                pltpu.sync_copy(acc.at[pl.ds(_NS * _RPT, _TAIL)],
                                out_hbm.at[c, pl.ds(_NS * _RPT, _TAIL)])


@functools.cache
def _sc_agg_kernel():
    mesh = plsc.VectorSubcoreMesh(core_axis_name="c", subcore_axis_name="s")
    return pl.kernel(
        _sc_agg_body,
        out_type=jax.ShapeDtypeStruct((_NC, _NPC, D), jnp.float32),
        mesh=mesh,
        scratch_types=[
            pltpu.VMEM((_CH, _C), jnp.int32),
            pltpu.VMEM((_CH, _C), jnp.int32),
            pltpu.VMEM((_CH, _C), jnp.float32),
            pltpu.VMEM((_C, D), jnp.float32),
            pltpu.VMEM((_C,), jnp.int32),
            pltpu.VMEM_SHARED((_ACC_ROWS, D), jnp.float32),
            pltpu.SemaphoreType.DMA,
        ],
    )


def _pad_tiles(a, fill):
    a = a.reshape(_NS, _EPT)
    a = jnp.pad(a, ((0, 0), (0, _EPT_PAD - _EPT)), constant_values=fill)
    return a.reshape(_NS, _CH, _C)


def kernel(x, edge_index, edge_weight, W1, W2):
    src = _pad_tiles(edge_index[0].astype(jnp.int32), 0)
    dst = _pad_tiles(edge_index[1].astype(jnp.int32), 0)
    w = _pad_tiles(edge_weight, 0.0)
    zeros = jnp.zeros((_ACC_ROWS, D), jnp.float32)

    agg = _sc_agg_kernel()
    p = agg(x, src, dst, w, zeros)
    y1 = p.reshape(N_NODES, D)
    q = agg(y1, src, dst, w, zeros)
    return _tc_fused_mm(q.reshape(N_NODES, D), W1, W2)
